# baseline (speedup 1.0000x reference)
import functools

import jax
import jax.numpy as jnp
import numpy as np
from jax import lax
from jax.experimental import pallas as pl
from jax.experimental.pallas import tpu as pltpu
from jax.experimental.pallas import tpu_sc as plsc

_N = 100000
_E = 1600000
_D_OUT = 128
_WIDTHS = [12, 6, 5, 6, 5, 1, 1, 1, 1]
_OFFS = np.concatenate([[0], np.cumsum(_WIDTHS)])
_NG = len(_WIDTHS)
_DIN = int(_OFFS[-1])
_GW = 16

_EROWS = _E // 128
_K = 6
_RPT = 390
_NTRIPS = _RPT // _K
_TAILBASE = 32 * _RPT
_ZROWS = _N // 16

_GBLK = 2048
_OBLK = 1024

_CDIM0 = (((0,), (1,)), ((), ()))
_CBOTH0 = (((0,), (0,)), ((), ()))


def _gates_body(x_ref, wg_ref, gt_ref):
    gt_ref[...] = lax.dot_general(wg_ref[...], x_ref[...], _CDIM0,
                                  preferred_element_type=jnp.float32)


def _out_body(x_ref, p0_ref, p1_ref, exp_ref, wf_ref, out_ref, attn_ref):
    pr = _OBLK // 8
    lp = p0_ref[...] + p1_ref[...]
    lp = jnp.clip(jnp.nan_to_num(lp), -80.0, 80.0)
    lane = lax.broadcasted_iota(jnp.int32, (pr, 128), 1)
    valid = lax.rem(lane, 16) < _NG
    e = jnp.where(valid, jnp.exp(lp), 0.0)
    blk16 = jnp.where(
        lax.broadcasted_iota(jnp.int32, (128, 128), 0) // 16
        == lax.broadcasted_iota(jnp.int32, (128, 128), 1) // 16,
        1.0, 0.0)
    ssum = jnp.dot(e, blk16, preferred_element_type=jnp.float32)
    ap = e / ssum
    attn_ref[...] = ap
    sub = lax.broadcasted_iota(jnp.int32, (_OBLK, 128), 0)
    lane2 = lax.broadcasted_iota(jnp.int32, (_OBLK, 128), 1)
    u = jnp.where(lane2 == sub // 8, 1.0, 0.0)
    y = jnp.dot(u, ap, preferred_element_type=jnp.float32)
    rowk = lax.rem(lax.broadcasted_iota(jnp.int32, (_OBLK, _DIN), 0), 8)
    ax = jnp.zeros((_OBLK, _DIN), jnp.float32)
    for k in range(8):
        axk = jnp.dot(y[:, 16 * k:16 * (k + 1)], exp_ref[...],
                      preferred_element_type=jnp.float32)
        ax = ax + jnp.where(rowk == k, axk, 0.0)
    out_ref[...] = jnp.dot(x_ref[...] * ax, wf_ref[...],
                           preferred_element_type=jnp.float32)


def _segment_sum_sc(g, edges, zrows):
    mesh = plsc.VectorSubcoreMesh(core_axis_name="c", subcore_axis_name="s")

    @functools.partial(
        pl.kernel,
        out_type=[jax.ShapeDtypeStruct((_N, _GW), jnp.float32),
                  jax.ShapeDtypeStruct((_N, _GW), jnp.float32)],
        mesh=mesh,
        scratch_types=[
            pltpu.VMEM_SHARED((_N, _GW), jnp.float32),
            pltpu.VMEM((2, _K * 128), jnp.int32),
            pltpu.VMEM((2, _K * 128), jnp.int32),
            pltpu.VMEM((2, _K, 128, _GW), jnp.float32),
            pltpu.SemaphoreType.DMA,
        ],
        compiler_params=pltpu.CompilerParams(use_tc_tiling_on_sc=False),
    )
    def seg_sum(g_hbm, e_hbm, z_hbm, out0_hbm, out1_hbm,
                acc, sidx, didx, rows, sem):
        cid = lax.axis_index("c")
        sid = lax.axis_index("s")
        zbase = sid * _ZROWS
        pltpu.sync_copy(z_hbm, acc.at[pl.ds(zbase, _ZROWS)])
        plsc.subcore_barrier()

        wid = cid * 16 + sid
        row0 = wid * _RPT

        def stage(par, rb):
            pltpu.sync_copy(e_hbm.at[0, pl.ds(rb * 128, _K * 128)],
                            sidx.at[par])
            pltpu.sync_copy(e_hbm.at[1, pl.ds(rb * 128, _K * 128)],
                            didx.at[par])
            for j in range(_K):
                pltpu.async_copy(
                    g_hbm.at[sidx.at[par, pl.ds(j * 128, 128)]],
                    rows.at[par, j], sem)

        stage(0, row0)

        def step(gi, carry):
            par = lax.rem(gi, 2)
            for j in range(_K):
                pltpu.make_async_copy(
                    g_hbm.at[sidx.at[par, pl.ds(j * 128, 128)]],
                    rows.at[par, j], sem).wait()

            @pl.when(gi + 1 < _NTRIPS)
            def _():
                stage(1 - par, row0 + (gi + 1) * _K)

            for j in range(_K):
                pltpu.sync_copy(
                    rows.at[par, j],
                    acc.at[didx.at[par, pl.ds(j * 128, 128)]], add=True)
            return carry

        lax.fori_loop(0, _NTRIPS, step, 0)

        @pl.when(wid < 20)
        def _():
            tb = (_TAILBASE + wid) * 128
            pltpu.sync_copy(e_hbm.at[0, pl.ds(tb, 128)],
                            sidx.at[0, pl.ds(0, 128)])
            pltpu.sync_copy(e_hbm.at[1, pl.ds(tb, 128)],
                            didx.at[0, pl.ds(0, 128)])
            pltpu.async_copy(g_hbm.at[sidx.at[0, pl.ds(0, 128)]],
                             rows.at[0, 0], sem).wait()
            pltpu.sync_copy(rows.at[0, 0],
                            acc.at[didx.at[0, pl.ds(0, 128)]], add=True)

        plsc.subcore_barrier()

        @pl.when(cid == 0)
        def _():
            pltpu.sync_copy(acc.at[pl.ds(zbase, _ZROWS)],
                            out0_hbm.at[pl.ds(zbase, _ZROWS)])

        @pl.when(cid == 1)
        def _():
            pltpu.sync_copy(acc.at[pl.ds(zbase, _ZROWS)],
                            out1_hbm.at[pl.ds(zbase, _ZROWS)])

    return seg_sum(g, edges, zrows)


def kernel(x, edge_index, batch, Wg, Wf):
    del batch

    wg_block = jnp.zeros((_DIN, _GW), jnp.float32)
    for i in range(_NG):
        wg_block = wg_block.at[_OFFS[i]:_OFFS[i + 1], i].set(Wg[i][:, 0])
    wf_block = jnp.concatenate(Wf, axis=0)

    expand = np.zeros((_GW, _DIN), np.float32)
    for i in range(_NG):
        expand[i, _OFFS[i]:_OFFS[i + 1]] = 1.0
    expand = jnp.asarray(expand)

    zrows = jnp.zeros((_ZROWS, _GW), jnp.float32)

    gt = pl.pallas_call(
        _gates_body,
        grid=(pl.cdiv(_N, _GBLK),),
        in_specs=[
            pl.BlockSpec((_GBLK, _DIN), lambda i: (i, 0)),
            pl.BlockSpec((_DIN, _GW), lambda i: (0, 0)),
        ],
        out_specs=pl.BlockSpec((_GW, _GBLK), lambda i: (0, i)),
        out_shape=jax.ShapeDtypeStruct((_GW, _N), jnp.float32),
    )(x, wg_block)

    p0, p1 = _segment_sum_sc(gt.T, edge_index, zrows)

    out, attn_p = pl.pallas_call(
        _out_body,
        grid=(pl.cdiv(_N, _OBLK),),
        in_specs=[
            pl.BlockSpec((_OBLK, _DIN), lambda i: (i, 0)),
            pl.BlockSpec((_OBLK // 8, 128), lambda i: (i, 0)),
            pl.BlockSpec((_OBLK // 8, 128), lambda i: (i, 0)),
            pl.BlockSpec((_GW, _DIN), lambda i: (0, 0)),
            pl.BlockSpec((_DIN, _D_OUT), lambda i: (0, 0)),
        ],
        out_specs=[
            pl.BlockSpec((_OBLK, _D_OUT), lambda i: (i, 0)),
            pl.BlockSpec((_OBLK // 8, 128), lambda i: (i, 0)),
        ],
        out_shape=[
            jax.ShapeDtypeStruct((_N, _D_OUT), jnp.float32),
            jax.ShapeDtypeStruct((_N // 8, 128), jnp.float32),
        ],
    )(x, p0.reshape(_N // 8, 128), p1.reshape(_N // 8, 128),
      expand, wf_block)

    attention = attn_p.reshape(_N, _GW)[:, :_NG][:, :, None]
    return out, attention


# speedup vs baseline: 1.1267x; 1.1267x over previous
import functools

import jax
import jax.numpy as jnp
import numpy as np
from jax import lax
from jax.experimental import pallas as pl
from jax.experimental.pallas import tpu as pltpu
from jax.experimental.pallas import tpu_sc as plsc

_N = 100000
_E = 1600000
_D_OUT = 128
_WIDTHS = [12, 6, 5, 6, 5, 1, 1, 1, 1]
_OFFS = np.concatenate([[0], np.cumsum(_WIDTHS)])
_NG = len(_WIDTHS)
_DIN = int(_OFFS[-1])
_GW = 16

_EROWS = _E // 128
_K = 6
_RPT = 390
_NTRIPS = _RPT // _K
_TAILBASE = 32 * _RPT
_ZROWS = _N // 16

_GBLK = 2048
_OBLK = 1024

_CDIM0 = (((0,), (1,)), ((), ()))
_CBOTH0 = (((0,), (0,)), ((), ()))


def _gates_body(x_ref, wg_ref, gt_ref):
    gt_ref[...] = lax.dot_general(wg_ref[...], x_ref[...], _CDIM0,
                                  preferred_element_type=jnp.float32)


def _out_body(x_ref, p0_ref, p1_ref, exp_ref, wf_ref, out_ref, attn_ref):
    lt = p0_ref[...] + p1_ref[...]
    gate = lax.broadcasted_iota(jnp.int32, lt.shape, 0)
    valid = gate < _NG
    lm = jnp.where(valid, lt, -1e30)
    m = jnp.max(lm, axis=0, keepdims=True)
    e = jnp.where(valid, jnp.exp(lm - m), 0.0)
    s = jnp.sum(e, axis=0, keepdims=True)
    at = e / s
    attn_ref[...] = at
    ax = lax.dot_general(at, exp_ref[...], _CBOTH0,
                         preferred_element_type=jnp.float32)
    out_ref[...] = jnp.dot(x_ref[...] * ax, wf_ref[...],
                           preferred_element_type=jnp.float32)


def _segment_sum_sc(g, edge3, zrows):
    mesh = plsc.VectorSubcoreMesh(core_axis_name="c", subcore_axis_name="s")

    @functools.partial(
        pl.kernel,
        out_type=[jax.ShapeDtypeStruct((_N, _GW), jnp.float32),
                  jax.ShapeDtypeStruct((_N, _GW), jnp.float32)],
        mesh=mesh,
        scratch_types=[
            pltpu.VMEM_SHARED((_N, _GW), jnp.float32),
            pltpu.VMEM((2, _K, 128), jnp.int32),
            pltpu.VMEM((2, _K, 128), jnp.int32),
            pltpu.VMEM((2, _K, 128, _GW), jnp.float32),
            pltpu.SemaphoreType.DMA,
        ],
        compiler_params=pltpu.CompilerParams(use_tc_tiling_on_sc=False),
    )
    def seg_sum(g_hbm, e_hbm, z_hbm, out0_hbm, out1_hbm,
                acc, sidx, didx, rows, sem):
        cid = lax.axis_index("c")
        sid = lax.axis_index("s")
        zbase = sid * _ZROWS
        pltpu.sync_copy(z_hbm, acc.at[pl.ds(zbase, _ZROWS)])
        plsc.subcore_barrier()

        wid = cid * 16 + sid
        row0 = wid * _RPT

        def stage(par, rb):
            pltpu.sync_copy(e_hbm.at[0, pl.ds(rb, _K)], sidx.at[par])
            pltpu.sync_copy(e_hbm.at[1, pl.ds(rb, _K)], didx.at[par])
            for j in range(_K):
                pltpu.async_copy(g_hbm.at[sidx.at[par, j]],
                                 rows.at[par, j], sem)

        stage(0, row0)

        def step(gi, carry):
            par = lax.rem(gi, 2)
            for j in range(_K):
                pltpu.make_async_copy(g_hbm.at[sidx.at[par, j]],
                                      rows.at[par, j], sem).wait()

            @pl.when(gi + 1 < _NTRIPS)
            def _():
                stage(1 - par, row0 + (gi + 1) * _K)

            for j in range(_K):
                pltpu.sync_copy(rows.at[par, j], acc.at[didx.at[par, j]],
                                add=True)
            return carry

        lax.fori_loop(0, _NTRIPS, step, 0)

        @pl.when(wid < 20)
        def _():
            pltpu.sync_copy(e_hbm.at[0, pl.ds(_TAILBASE + wid, 1)],
                            sidx.at[0, pl.ds(0, 1)])
            pltpu.sync_copy(e_hbm.at[1, pl.ds(_TAILBASE + wid, 1)],
                            didx.at[0, pl.ds(0, 1)])
            pltpu.async_copy(g_hbm.at[sidx.at[0, 0]],
                             rows.at[0, 0], sem).wait()
            pltpu.sync_copy(rows.at[0, 0], acc.at[didx.at[0, 0]],
                            add=True)

        plsc.subcore_barrier()

        @pl.when(cid == 0)
        def _():
            pltpu.sync_copy(acc.at[pl.ds(zbase, _ZROWS)],
                            out0_hbm.at[pl.ds(zbase, _ZROWS)])

        @pl.when(cid == 1)
        def _():
            pltpu.sync_copy(acc.at[pl.ds(zbase, _ZROWS)],
                            out1_hbm.at[pl.ds(zbase, _ZROWS)])

    return seg_sum(g, edge3, zrows)


def kernel(x, edge_index, batch, Wg, Wf):
    del batch

    wg_block = jnp.zeros((_DIN, _GW), jnp.float32)
    for i in range(_NG):
        wg_block = wg_block.at[_OFFS[i]:_OFFS[i + 1], i].set(Wg[i][:, 0])
    wf_block = jnp.concatenate(Wf, axis=0)

    expand = np.zeros((_GW, _DIN), np.float32)
    for i in range(_NG):
        expand[i, _OFFS[i]:_OFFS[i + 1]] = 1.0
    expand = jnp.asarray(expand)

    edge3 = edge_index.reshape(2, _EROWS, 128)
    zrows = jnp.zeros((_ZROWS, _GW), jnp.float32)

    gt = pl.pallas_call(
        _gates_body,
        grid=(pl.cdiv(_N, _GBLK),),
        in_specs=[
            pl.BlockSpec((_GBLK, _DIN), lambda i: (i, 0)),
            pl.BlockSpec((_DIN, _GW), lambda i: (0, 0)),
        ],
        out_specs=pl.BlockSpec((_GW, _GBLK), lambda i: (0, i)),
        out_shape=jax.ShapeDtypeStruct((_GW, _N), jnp.float32),
    )(x, wg_block)

    p0, p1 = _segment_sum_sc(gt.T, edge3, zrows)

    out, attn_t = pl.pallas_call(
        _out_body,
        grid=(pl.cdiv(_N, _OBLK),),
        in_specs=[
            pl.BlockSpec((_OBLK, _DIN), lambda i: (i, 0)),
            pl.BlockSpec((_GW, _OBLK), lambda i: (0, i)),
            pl.BlockSpec((_GW, _OBLK), lambda i: (0, i)),
            pl.BlockSpec((_GW, _DIN), lambda i: (0, 0)),
            pl.BlockSpec((_DIN, _D_OUT), lambda i: (0, 0)),
        ],
        out_specs=[
            pl.BlockSpec((_OBLK, _D_OUT), lambda i: (i, 0)),
            pl.BlockSpec((_GW, _OBLK), lambda i: (0, i)),
        ],
        out_shape=[
            jax.ShapeDtypeStruct((_N, _D_OUT), jnp.float32),
            jax.ShapeDtypeStruct((_GW, _N), jnp.float32),
        ],
    )(x, p0.T, p1.T, expand, wf_block)

    return out, attn_t[:_NG].T[:, :, None]
